# baseline (speedup 1.0000x reference)
import functools
import math

import jax
import jax.numpy as jnp
from jax import lax
from jax.experimental import pallas as pl
from jax.experimental.pallas import tpu as pltpu
from jax.experimental.pallas import tpu_sc as plsc

D_MODEL_K = 768
D_FF_K = 2048
E_K = 64
T_K = 2048
CAP_K = int(math.ceil(1.25 * 1 * T_K / E_K))
NROWS_K = E_K * CAP_K + 8
TRASH_K = E_K * CAP_K

NC, NS = 2, 16
NW = NC * NS
TPW = T_K // NW


def _router_body(x_ref, wg_ref, gate_ref, sidx_ref, gidx_ref):
    x = x_ref[...]
    logits = jnp.dot(x, wg_ref[...], preferred_element_type=jnp.float32)
    m = jnp.max(logits, axis=1, keepdims=True)
    p = jnp.exp(logits - m)
    probs = p / jnp.sum(p, axis=1, keepdims=True)
    gate = jnp.max(probs, axis=1)
    lanes = lax.broadcasted_iota(jnp.int32, (T_K, E_K), 1)
    eidx = jnp.min(jnp.where(probs == gate[:, None], lanes, E_K), axis=1)
    onehot = (lanes == eidx[:, None]).astype(jnp.int32)
    csum = onehot
    s = 1
    while s < T_K:
        csum = csum + jnp.concatenate(
            [jnp.zeros((s, E_K), jnp.int32), csum[:-s]], axis=0)
        s *= 2
    pos = jnp.sum(csum * onehot, axis=1) - 1
    keep = pos < CAP_K
    pos_c = jnp.clip(pos, 0, CAP_K - 1)
    gidx = eidx * CAP_K + pos_c
    gate_ref[...] = jnp.where(keep, gate, 0.0)[:, None]
    sidx_ref[...] = jnp.where(keep, gidx, TRASH_K)
    gidx_ref[...] = gidx


def _router(x, wg):
    return pl.pallas_call(
        _router_body,
        out_shape=(
            jax.ShapeDtypeStruct((T_K, 1), jnp.float32),
            jax.ShapeDtypeStruct((T_K,), jnp.int32),
            jax.ShapeDtypeStruct((T_K,), jnp.int32),
        ),
    )(x, wg)


@functools.lru_cache(maxsize=None)
def _make_dispatch():
    mesh = plsc.VectorSubcoreMesh(core_axis_name="c", subcore_axis_name="s")

    @functools.partial(
        pl.kernel,
        out_type=jax.ShapeDtypeStruct((NROWS_K, D_MODEL_K), jnp.float32),
        mesh=mesh,
        scratch_types=[
            pltpu.VMEM((TPW,), jnp.int32),
            pltpu.VMEM((TPW, D_MODEL_K), jnp.float32),
            pltpu.SemaphoreType.DMA,
        ],
    )
    def _dispatch(x_hbm, sidx_hbm, buf_hbm, idx_v, rows_v, sem):
        wid = lax.axis_index("s") * NC + lax.axis_index("c")
        base = wid * TPW
        pltpu.sync_copy(sidx_hbm.at[pl.ds(base, TPW)], idx_v)
        pltpu.sync_copy(x_hbm.at[pl.ds(base, TPW)], rows_v)
        pltpu.async_copy(rows_v, buf_hbm.at[idx_v], sem).wait()

    return _dispatch


FF_BLK = 1024
N_FF_BLK = D_FF_K // FF_BLK


def _ffn_body(buf_ref, w1_ref, b1_ref, w2_ref, b2_ref, y_ref):
    f = pl.program_id(1)
    xb = buf_ref[...]
    h = jnp.dot(xb, w1_ref[0], preferred_element_type=jnp.float32)
    h = jax.nn.gelu(h + b1_ref[0])
    yp = jnp.dot(h, w2_ref[0], preferred_element_type=jnp.float32)

    @pl.when(f == 0)
    def _():
        y_ref[...] = yp + b2_ref[0]

    @pl.when(f != 0)
    def _():
        y_ref[...] = y_ref[...] + yp


def _ffn(buf, w1, b1, w2, b2):
    return pl.pallas_call(
        _ffn_body,
        grid=(E_K, N_FF_BLK),
        in_specs=[
            pl.BlockSpec((CAP_K, D_MODEL_K), lambda e, f: (e, 0)),
            pl.BlockSpec((1, D_MODEL_K, FF_BLK), lambda e, f: (e, 0, f)),
            pl.BlockSpec((1, 1, FF_BLK), lambda e, f: (e, 0, f)),
            pl.BlockSpec((1, FF_BLK, D_MODEL_K), lambda e, f: (e, f, 0)),
            pl.BlockSpec((1, 1, D_MODEL_K), lambda e, f: (e, 0, 0)),
        ],
        out_specs=pl.BlockSpec((CAP_K, D_MODEL_K), lambda e, f: (e, 0)),
        out_shape=jax.ShapeDtypeStruct((E_K * CAP_K, D_MODEL_K), jnp.float32),
    )(buf, w1, b1[:, None, :], w2, b2[:, None, :])


@functools.lru_cache(maxsize=None)
def _make_combine():
    mesh = plsc.VectorSubcoreMesh(core_axis_name="c", subcore_axis_name="s")

    @functools.partial(
        pl.kernel,
        out_type=jax.ShapeDtypeStruct((T_K, D_MODEL_K), jnp.float32),
        mesh=mesh,
        scratch_types=[
            pltpu.VMEM((TPW,), jnp.int32),
            pltpu.VMEM((TPW, D_MODEL_K), jnp.float32),
            pltpu.SemaphoreType.DMA,
        ],
    )
    def _combine(y_hbm, gidx_hbm, g_hbm, idx_v, rows_v, sem):
        wid = lax.axis_index("s") * NC + lax.axis_index("c")
        base = wid * TPW
        pltpu.sync_copy(gidx_hbm.at[pl.ds(base, TPW)], idx_v)
        pltpu.async_copy(y_hbm.at[idx_v], rows_v, sem).wait()
        pltpu.sync_copy(rows_v, g_hbm.at[pl.ds(base, TPW)])

    return _combine


def _scale_body(g_ref, gate_ref, out_ref):
    out_ref[...] = g_ref[...] * gate_ref[...]


def _scale(g, gate):
    return pl.pallas_call(
        _scale_body,
        grid=(8,),
        in_specs=[
            pl.BlockSpec((T_K // 8, D_MODEL_K), lambda i: (i, 0)),
            pl.BlockSpec((T_K // 8, 1), lambda i: (i, 0)),
        ],
        out_specs=pl.BlockSpec((T_K // 8, D_MODEL_K), lambda i: (i, 0)),
        out_shape=jax.ShapeDtypeStruct((T_K, D_MODEL_K), jnp.float32),
    )(g, gate)


@jax.jit
def kernel(hidden_states, wg, w1, b1, w2, b2):
    B, S, D = hidden_states.shape
    x = hidden_states.reshape(-1, D)
    gate, sidx, gidx = _router(x, wg)
    buf = _make_dispatch()(x, sidx)
    y = _ffn(buf, w1, b1, w2, b2)
    g = _make_combine()(y, gidx)
    out = _scale(g, gate)
    return out.reshape(B, S, D)


# speedup vs baseline: 1.0475x; 1.0475x over previous
import functools
import math

import jax
import jax.numpy as jnp
from jax import lax
from jax.experimental import pallas as pl
from jax.experimental.pallas import tpu as pltpu
from jax.experimental.pallas import tpu_sc as plsc

D_MODEL_K = 768
D_FF_K = 2048
E_K = 64
T_K = 2048
CAP_K = int(math.ceil(1.25 * 1 * T_K / E_K))
NROWS_K = E_K * CAP_K + 8
TRASH_K = E_K * CAP_K

NC, NS = 2, 16
NW = NC * NS
TPW = T_K // NW


def _router_body(x_ref, wg_ref, gate_ref, sidx_ref, gidx_ref):
    x = x_ref[...]
    logits = jnp.dot(x, wg_ref[...], preferred_element_type=jnp.float32)
    m = jnp.max(logits, axis=1, keepdims=True)
    p = jnp.exp(logits - m)
    probs = p / jnp.sum(p, axis=1, keepdims=True)
    gate = jnp.max(probs, axis=1)
    lanes = lax.broadcasted_iota(jnp.int32, (T_K, E_K), 1)
    eidx = jnp.min(jnp.where(probs == gate[:, None], lanes, E_K), axis=1)
    onehot = (lanes == eidx[:, None]).astype(jnp.int32)
    csum = onehot
    s = 1
    while s < T_K:
        csum = csum + jnp.concatenate(
            [jnp.zeros((s, E_K), jnp.int32), csum[:-s]], axis=0)
        s *= 2
    pos = jnp.sum(csum * onehot, axis=1) - 1
    keep = pos < CAP_K
    pos_c = jnp.clip(pos, 0, CAP_K - 1)
    gidx = eidx * CAP_K + pos_c
    gate_ref[...] = jnp.where(keep, gate, 0.0)
    sidx_ref[...] = jnp.where(keep, gidx, TRASH_K)
    gidx_ref[...] = gidx


def _router(x, wg):
    return pl.pallas_call(
        _router_body,
        out_shape=(
            jax.ShapeDtypeStruct((T_K,), jnp.float32),
            jax.ShapeDtypeStruct((T_K,), jnp.int32),
            jax.ShapeDtypeStruct((T_K,), jnp.int32),
        ),
    )(x, wg)


@functools.lru_cache(maxsize=None)
def _make_dispatch():
    mesh = plsc.VectorSubcoreMesh(core_axis_name="c", subcore_axis_name="s")

    @functools.partial(
        pl.kernel,
        out_type=jax.ShapeDtypeStruct((NROWS_K, D_MODEL_K), jnp.float32),
        mesh=mesh,
        scratch_types=[
            pltpu.VMEM((TPW,), jnp.int32),
            pltpu.VMEM((TPW, D_MODEL_K), jnp.float32),
            pltpu.SemaphoreType.DMA,
        ],
    )
    def _dispatch(x_hbm, sidx_hbm, buf_hbm, idx_v, rows_v, sem):
        wid = lax.axis_index("s") * NC + lax.axis_index("c")
        base = wid * TPW
        pltpu.sync_copy(sidx_hbm.at[pl.ds(base, TPW)], idx_v)
        pltpu.sync_copy(x_hbm.at[pl.ds(base, TPW)], rows_v)
        pltpu.async_copy(rows_v, buf_hbm.at[idx_v], sem).wait()

    return _dispatch


FF_BLK = 2048
N_FF_BLK = D_FF_K // FF_BLK


def _ffn_body(buf_ref, w1_ref, b1_ref, w2_ref, b2_ref, y_ref):
    f = pl.program_id(1)
    xb = buf_ref[...]
    h = jnp.dot(xb, w1_ref[0], preferred_element_type=jnp.float32)
    h = jax.nn.gelu(h + b1_ref[0])
    yp = jnp.dot(h, w2_ref[0], preferred_element_type=jnp.float32)

    @pl.when(f == 0)
    def _():
        y_ref[...] = yp + b2_ref[0]

    @pl.when(f != 0)
    def _():
        y_ref[...] = y_ref[...] + yp


def _ffn(buf, w1, b1, w2, b2):
    return pl.pallas_call(
        _ffn_body,
        grid=(E_K, N_FF_BLK),
        in_specs=[
            pl.BlockSpec((CAP_K, D_MODEL_K), lambda e, f: (e, 0)),
            pl.BlockSpec((1, D_MODEL_K, FF_BLK), lambda e, f: (e, 0, f)),
            pl.BlockSpec((1, 1, FF_BLK), lambda e, f: (e, 0, f)),
            pl.BlockSpec((1, FF_BLK, D_MODEL_K), lambda e, f: (e, f, 0)),
            pl.BlockSpec((1, 1, D_MODEL_K), lambda e, f: (e, 0, 0)),
        ],
        out_specs=pl.BlockSpec((CAP_K, D_MODEL_K), lambda e, f: (e, 0)),
        out_shape=jax.ShapeDtypeStruct((E_K * CAP_K, D_MODEL_K), jnp.float32),
    )(buf, w1, b1[:, None, :], w2, b2[:, None, :])


@functools.lru_cache(maxsize=None)
def _make_combine():
    mesh = plsc.VectorSubcoreMesh(core_axis_name="c", subcore_axis_name="s")

    @functools.partial(
        pl.kernel,
        out_type=jax.ShapeDtypeStruct((T_K, D_MODEL_K), jnp.float32),
        mesh=mesh,
        scratch_types=[
            pltpu.VMEM((TPW,), jnp.int32),
            pltpu.VMEM((TPW,), jnp.float32),
            pltpu.VMEM((TPW, D_MODEL_K), jnp.float32),
            pltpu.SemaphoreType.DMA,
        ],
    )
    def _combine(y_hbm, gidx_hbm, gate_hbm, out_hbm, idx_v, gate_v, rows_v, sem):
        wid = lax.axis_index("s") * NC + lax.axis_index("c")
        base = wid * TPW
        pltpu.sync_copy(gidx_hbm.at[pl.ds(base, TPW)], idx_v)
        pltpu.sync_copy(gate_hbm.at[pl.ds(base, TPW)], gate_v)
        pltpu.async_copy(y_hbm.at[idx_v], rows_v, sem).wait()

        dn = lax.GatherDimensionNumbers(
            offset_dims=(), collapsed_slice_dims=(0,), start_index_map=(0,))

        def _row(i, carry):
            g16 = gate_v[pl.ds((i // 16) * 16, 16)]
            g = lax.gather(g16, jnp.full((16, 1), i % 16, jnp.int32), dn,
                           slice_sizes=(1,),
                           mode=lax.GatherScatterMode.PROMISE_IN_BOUNDS)
            for j in range(D_MODEL_K // 16):
                rows_v[i, pl.ds(j * 16, 16)] = rows_v[i, pl.ds(j * 16, 16)] * g
            return carry

        lax.fori_loop(0, TPW, _row, 0)
        pltpu.sync_copy(rows_v, out_hbm.at[pl.ds(base, TPW)])

    return _combine


def _scale_body(g_ref, gate_ref, out_ref):
    out_ref[...] = g_ref[...] * gate_ref[...]


def _scale(g, gate):
    return pl.pallas_call(
        _scale_body,
        grid=(8,),
        in_specs=[
            pl.BlockSpec((T_K // 8, D_MODEL_K), lambda i: (i, 0)),
            pl.BlockSpec((T_K // 8, 1), lambda i: (i, 0)),
        ],
        out_specs=pl.BlockSpec((T_K // 8, D_MODEL_K), lambda i: (i, 0)),
        out_shape=jax.ShapeDtypeStruct((T_K, D_MODEL_K), jnp.float32),
    )(g, gate)


@jax.jit
def kernel(hidden_states, wg, w1, b1, w2, b2):
    B, S, D = hidden_states.shape
    x = hidden_states.reshape(-1, D)
    gate, sidx, gidx = _router(x, wg)
    buf = _make_dispatch()(x, sidx)
    y = _ffn(buf, w1, b1, w2, b2)
    out = _make_combine()(y, gidx, gate)
    return out.reshape(B, S, D)


# speedup vs baseline: 1.0495x; 1.0019x over previous
import functools
import math

import jax
import jax.numpy as jnp
from jax import lax
from jax.experimental import pallas as pl
from jax.experimental.pallas import tpu as pltpu
from jax.experimental.pallas import tpu_sc as plsc

D_MODEL_K = 768
D_FF_K = 2048
E_K = 64
T_K = 2048
CAP_K = int(math.ceil(1.25 * 1 * T_K / E_K))
NROWS_K = E_K * CAP_K + 8
TRASH_K = E_K * CAP_K

NC, NS = 2, 16
NW = NC * NS
TPW = T_K // NW


def _router_body(x_ref, wg_ref, gate_ref, sidx_ref, gidx_ref):
    x = x_ref[0]
    logits = jnp.dot(x, wg_ref[...], preferred_element_type=jnp.float32)
    m = jnp.max(logits, axis=1, keepdims=True)
    p = jnp.exp(logits - m)
    probs = p / jnp.sum(p, axis=1, keepdims=True)
    gate = jnp.max(probs, axis=1)
    lanes = lax.broadcasted_iota(jnp.int32, (T_K, E_K), 1)
    eidx = jnp.min(jnp.where(probs == gate[:, None], lanes, E_K), axis=1)
    onehot = (lanes == eidx[:, None]).astype(jnp.int32)
    csum = onehot
    s = 1
    while s < T_K:
        csum = csum + jnp.concatenate(
            [jnp.zeros((s, E_K), jnp.int32), csum[:-s]], axis=0)
        s *= 2
    pos = jnp.sum(csum * onehot, axis=1) - 1
    keep = pos < CAP_K
    pos_c = jnp.clip(pos, 0, CAP_K - 1)
    gidx = eidx * CAP_K + pos_c
    gate_ref[...] = jnp.where(keep, gate, 0.0)
    sidx_ref[...] = jnp.where(keep, gidx, TRASH_K)
    gidx_ref[...] = gidx


def _router(hs, wg):
    return pl.pallas_call(
        _router_body,
        out_shape=(
            jax.ShapeDtypeStruct((T_K,), jnp.float32),
            jax.ShapeDtypeStruct((T_K,), jnp.int32),
            jax.ShapeDtypeStruct((T_K,), jnp.int32),
        ),
    )(hs, wg)


@functools.lru_cache(maxsize=None)
def _make_dispatch():
    mesh = plsc.VectorSubcoreMesh(core_axis_name="c", subcore_axis_name="s")

    @functools.partial(
        pl.kernel,
        out_type=jax.ShapeDtypeStruct((NROWS_K, D_MODEL_K), jnp.float32),
        mesh=mesh,
        scratch_types=[
            pltpu.VMEM((TPW,), jnp.int32),
            pltpu.VMEM((TPW, D_MODEL_K), jnp.float32),
            pltpu.SemaphoreType.DMA,
        ],
    )
    def _dispatch(x_hbm, sidx_hbm, buf_hbm, idx_v, rows_v, sem):
        wid = lax.axis_index("s") * NC + lax.axis_index("c")
        base = wid * TPW
        pltpu.sync_copy(sidx_hbm.at[pl.ds(base, TPW)], idx_v)
        pltpu.sync_copy(x_hbm.at[0, pl.ds(base, TPW)], rows_v)
        pltpu.async_copy(rows_v, buf_hbm.at[idx_v], sem).wait()

    return _dispatch


FF_BLK = 2048
N_FF_BLK = D_FF_K // FF_BLK


def _ffn_body(buf_ref, w1_ref, b1_ref, w2_ref, b2_ref, y_ref):
    e = pl.program_id(0)
    f = pl.program_id(1)
    xb = buf_ref[...]
    h = jnp.dot(xb, w1_ref[0], preferred_element_type=jnp.float32)
    h = jax.nn.gelu(h + b1_ref[pl.ds(e, 1)])
    yp = jnp.dot(h, w2_ref[0], preferred_element_type=jnp.float32)

    @pl.when(f == 0)
    def _():
        y_ref[...] = yp + b2_ref[pl.ds(e, 1)]

    @pl.when(f != 0)
    def _():
        y_ref[...] = y_ref[...] + yp


def _ffn(buf, w1, b1, w2, b2):
    return pl.pallas_call(
        _ffn_body,
        grid=(E_K, N_FF_BLK),
        in_specs=[
            pl.BlockSpec((CAP_K, D_MODEL_K), lambda e, f: (e, 0)),
            pl.BlockSpec((1, D_MODEL_K, FF_BLK), lambda e, f: (e, 0, f)),
            pl.BlockSpec((E_K, D_FF_K), lambda e, f: (0, 0)),
            pl.BlockSpec((1, FF_BLK, D_MODEL_K), lambda e, f: (e, f, 0)),
            pl.BlockSpec((E_K, D_MODEL_K), lambda e, f: (0, 0)),
        ],
        out_specs=pl.BlockSpec((CAP_K, D_MODEL_K), lambda e, f: (e, 0)),
        out_shape=jax.ShapeDtypeStruct((E_K * CAP_K, D_MODEL_K), jnp.float32),
    )(buf, w1, b1, w2, b2)


@functools.lru_cache(maxsize=None)
def _make_combine():
    mesh = plsc.VectorSubcoreMesh(core_axis_name="c", subcore_axis_name="s")

    @functools.partial(
        pl.kernel,
        out_type=jax.ShapeDtypeStruct((T_K, D_MODEL_K), jnp.float32),
        mesh=mesh,
        scratch_types=[
            pltpu.VMEM((TPW,), jnp.int32),
            pltpu.VMEM((TPW,), jnp.float32),
            pltpu.VMEM((TPW, D_MODEL_K), jnp.float32),
            pltpu.SemaphoreType.DMA,
        ],
    )
    def _combine(y_hbm, gidx_hbm, gate_hbm, out_hbm, idx_v, gate_v, rows_v, sem):
        wid = lax.axis_index("s") * NC + lax.axis_index("c")
        base = wid * TPW
        pltpu.sync_copy(gidx_hbm.at[pl.ds(base, TPW)], idx_v)
        pltpu.sync_copy(gate_hbm.at[pl.ds(base, TPW)], gate_v)
        pltpu.async_copy(y_hbm.at[idx_v], rows_v, sem).wait()

        dn = lax.GatherDimensionNumbers(
            offset_dims=(), collapsed_slice_dims=(0,), start_index_map=(0,))

        def _row(i, carry):
            g16 = gate_v[pl.ds((i // 16) * 16, 16)]
            g = lax.gather(g16, jnp.full((16, 1), i % 16, jnp.int32), dn,
                           slice_sizes=(1,),
                           mode=lax.GatherScatterMode.PROMISE_IN_BOUNDS)
            for j in range(D_MODEL_K // 16):
                rows_v[i, pl.ds(j * 16, 16)] = rows_v[i, pl.ds(j * 16, 16)] * g
            return carry

        lax.fori_loop(0, TPW, _row, 0)
        pltpu.sync_copy(rows_v, out_hbm.at[pl.ds(base, TPW)])

    return _combine


def _scale_body(g_ref, gate_ref, out_ref):
    out_ref[...] = g_ref[...] * gate_ref[...]


def _scale(g, gate):
    return pl.pallas_call(
        _scale_body,
        grid=(8,),
        in_specs=[
            pl.BlockSpec((T_K // 8, D_MODEL_K), lambda i: (i, 0)),
            pl.BlockSpec((T_K // 8, 1), lambda i: (i, 0)),
        ],
        out_specs=pl.BlockSpec((T_K // 8, D_MODEL_K), lambda i: (i, 0)),
        out_shape=jax.ShapeDtypeStruct((T_K, D_MODEL_K), jnp.float32),
    )(g, gate)


@jax.jit
def kernel(hidden_states, wg, w1, b1, w2, b2):
    B, S, D = hidden_states.shape
    gate, sidx, gidx = _router(hidden_states, wg)
    buf = _make_dispatch()(hidden_states, sidx)
    y = _ffn(buf, w1, b1, w2, b2)
    out = _make_combine()(y, gidx, gate)
    return out.reshape(B, S, D)
